# baseline (speedup 1.0000x reference)
import functools

import jax
import jax.numpy as jnp
from jax import lax
from jax.experimental import pallas as pl
from jax.experimental.pallas import tpu as pltpu
from jax.experimental.pallas import tpu_sc as plsc

N_FACTORS = 64
BATCH = 16384
NC, NS, L = 2, 16, 16
NW = NC * NS
B_PER_W = BATCH // NW
CHUNK = 128
N_CHUNKS = B_PER_W // CHUNK
GROUPS = B_PER_W // L


def _sc_body(u_hbm, m_hbm, uidx_hbm, midx_hbm, out_hbm,
             uidx_v, midx_v, u_rows, m_rows, out_v, sem):
    wid = lax.axis_index("s") * NC + lax.axis_index("c")
    base = wid * B_PER_W

    pltpu.sync_copy(uidx_hbm.at[wid], uidx_v)
    pltpu.sync_copy(midx_hbm.at[wid], midx_v)

    copies = []
    for c in range(N_CHUNKS):
        copies.append(pltpu.make_async_copy(
            u_hbm.at[uidx_v.at[c]], u_rows.at[pl.ds(c * CHUNK, CHUNK)], sem))
        copies.append(pltpu.make_async_copy(
            m_hbm.at[midx_v.at[c]], m_rows.at[pl.ds(c * CHUNK, CHUNK)], sem))
    for cp in copies:
        cp.start()
    for cp in copies:
        cp.wait()

    lane = lax.iota(jnp.int32, L)

    def group(g, carry):
        row = g * L + lane
        acc = jnp.zeros((L,), jnp.float32)
        for d in range(N_FACTORS):
            col = jnp.full((L,), d, jnp.int32)
            uu = plsc.load_gather(u_rows, [row, col])
            mm = plsc.load_gather(m_rows, [row, col])
            acc = acc + uu * mm
        out_v[pl.ds(g * L, L)] = acc
        return carry

    lax.fori_loop(0, GROUPS, group, 0)

    pltpu.sync_copy(out_v, out_hbm.at[pl.ds(base, B_PER_W)])


@jax.jit
def kernel(data, user_factors, movie_factors):
    uidx = data[:, 0].reshape(NW, N_CHUNKS, CHUNK)
    midx = data[:, 1].reshape(NW, N_CHUNKS, CHUNK)
    mesh = plsc.VectorSubcoreMesh(core_axis_name="c", subcore_axis_name="s")
    f = pl.kernel(
        _sc_body,
        out_type=jax.ShapeDtypeStruct((BATCH,), jnp.float32),
        mesh=mesh,
        scratch_types=[
            pltpu.VMEM((N_CHUNKS, CHUNK), jnp.int32),
            pltpu.VMEM((N_CHUNKS, CHUNK), jnp.int32),
            pltpu.VMEM((B_PER_W, N_FACTORS), jnp.float32),
            pltpu.VMEM((B_PER_W, N_FACTORS), jnp.float32),
            pltpu.VMEM((B_PER_W,), jnp.float32),
            pltpu.SemaphoreType.DMA,
        ],
        compiler_params=pltpu.CompilerParams(
            needs_layout_passes=False, use_tc_tiling_on_sc=False),
    )
    return f(user_factors, movie_factors, uidx, midx)
